# baseline (speedup 1.0000x reference)
import functools
import math

import numpy as np
import jax
import jax.numpy as jnp
from jax import lax
from jax.experimental import pallas as pl
from jax.experimental.pallas import tpu as pltpu
from jax.experimental.pallas import tpu_sc as plsc

_VOCAB = 100000
_D = 1024
_B = 4
_L = 2048
_NC, _NS = 2, 16
_NW = _NC * _NS
_PPW = _L // _NW
_ROWS = _B * _L
_CHUNK = 16
_NGROUP = _PPW // _CHUNK
_NCHUNK = _NGROUP * _B
_LANES = 16
_NBUF = 5


_NROT = _L // _CHUNK


def _pe_consts() -> np.ndarray:
    div = np.exp(np.arange(0, _D, 2).astype(np.float32)
                 * (-math.log(10000.0) / _D))
    pos = np.arange(_CHUNK, dtype=np.float32)[:, None]
    pe16 = np.zeros((_CHUNK, _D), dtype=np.float32)
    pe16[:, 0::2] = np.sin(pos * div)
    pe16[:, 1::2] = np.cos(pos * div)
    pe16s = np.zeros_like(pe16)
    pe16s[:, 0::2] = pe16[:, 1::2]
    pe16s[:, 1::2] = pe16[:, 0::2]
    k = (np.arange(_NROT, dtype=np.float32) * _CHUNK)[:, None]
    rot_a = np.zeros((_NROT, _D), dtype=np.float32)
    rot_a[:, 0::2] = np.cos(k * div)
    rot_a[:, 1::2] = rot_a[:, 0::2]
    rot_b = np.zeros((_NROT, _D), dtype=np.float32)
    rot_b[:, 0::2] = np.sin(k * div)
    rot_b[:, 1::2] = -rot_b[:, 0::2]
    return np.concatenate([pe16, pe16s, rot_a, rot_b], axis=0)


_CONSTS = _pe_consts()


def _embed_body(x_hbm, pe_hbm, table_hbm, out_hbm,
                idx_v, pv0, pv1, gb0, gb1, gb2, gb3, gb4,
                psem0, psem1, gsem0, gsem1, gsem2, gsem3, gsem4,
                ssem0, ssem1, ssem2, ssem3, ssem4):
    pe_v = (pv0, pv1)
    gbuf = (gb0, gb1, gb2, gb3, gb4)
    psem = (psem0, psem1)
    gsem = (gsem0, gsem1, gsem2, gsem3, gsem4)
    ssem = (ssem0, ssem1, ssem2, ssem3, ssem4)

    wid = lax.axis_index("s") * _NC + lax.axis_index("c")
    p0 = wid * _PPW

    for b in range(_B):
        pltpu.sync_copy(x_hbm.at[b, pl.ds(p0, _PPW)],
                        idx_v.at[pl.ds(b * _PPW, _PPW)])

    def start_gather(c):
        g, bt = divmod(c, _B)
        return pltpu.async_copy(
            table_hbm.at[idx_v.at[pl.ds(bt * _PPW + g * _CHUNK, _CHUNK)]],
            gbuf[c % _NBUF], gsem[c % _NBUF])

    def start_pe(g):
        return pltpu.async_copy(
            pe_hbm.at[pl.ds(p0 + g * _CHUNK, _CHUNK)], pe_v[g % 2],
            psem[g % 2])

    gathers = {c: start_gather(c) for c in range(_NBUF - 2)}
    pes = {0: start_pe(0), 1: start_pe(1)}
    stores = {}

    for c in range(_NCHUNK):
        g, bt = divmod(c, _B)
        if bt == 0:
            if 1 <= g <= _NGROUP - 2:
                pes[g + 1] = start_pe(g + 1)
            pes.pop(g).wait()
        gathers.pop(c).wait()

        gb, pv = gbuf[c % _NBUF], pe_v[g % 2]

        @plsc.parallel_loop(0, _CHUNK * _D // (2 * _LANES), 1, unroll=4)
        def add_body(i):
            r = lax.shift_right_logical(i, 5)
            c0 = pl.multiple_of(
                lax.bitwise_and(i, _D // (2 * _LANES) - 1) * 2 * _LANES,
                2 * _LANES)
            s1 = pl.ds(c0, _LANES)
            s2 = pl.ds(c0 + _LANES, _LANES)
            plsc.addupdate(gb.at[r, s1], pv[r, s1])
            plsc.addupdate(gb.at[r, s2], pv[r, s2])

        stores[c] = pltpu.async_copy(
            gb, out_hbm.at[pl.ds(bt * _L + p0 + g * _CHUNK, _CHUNK)],
            ssem[c % _NBUF])
        if c + _NBUF - 2 < _NCHUNK:
            if c >= 2:
                stores.pop(c - 2).wait()
            gathers[c + _NBUF - 2] = start_gather(c + _NBUF - 2)

    for c in sorted(stores):
        stores.pop(c).wait()


def _pe_tc_body(pe16_ref, pe16s_ref, ra_ref, rb_ref, o_ref):
    o_ref[...] = pe16_ref[...] * ra_ref[0] + pe16s_ref[...] * rb_ref[0]


@jax.jit
def _embed(x, table):
    consts = jnp.asarray(_CONSTS)
    pe = pl.pallas_call(
        _pe_tc_body,
        grid=(_NROT,),
        in_specs=[
            pl.BlockSpec((_CHUNK, _D), lambda k: (0, 0)),
            pl.BlockSpec((_CHUNK, _D), lambda k: (0, 0)),
            pl.BlockSpec((1, 1, _D), lambda k: (k, 0, 0)),
            pl.BlockSpec((1, 1, _D), lambda k: (k, 0, 0)),
        ],
        out_specs=pl.BlockSpec((_CHUNK, _D), lambda k: (k, 0)),
        out_shape=jax.ShapeDtypeStruct((_L, _D), jnp.float32),
    )(consts[0:_CHUNK], consts[_CHUNK:2 * _CHUNK],
      consts[2 * _CHUNK:2 * _CHUNK + _NROT].reshape(_NROT, 1, _D),
      consts[2 * _CHUNK + _NROT:].reshape(_NROT, 1, _D))
    mesh = plsc.VectorSubcoreMesh(core_axis_name="c", subcore_axis_name="s")
    f = functools.partial(
        pl.kernel,
        mesh=mesh,
        out_type=jax.ShapeDtypeStruct((_ROWS, _D), jnp.float32),
        scratch_types=[
            pltpu.VMEM((_B * _PPW,), jnp.int32),
            pltpu.VMEM((_CHUNK, _D), jnp.float32),
            pltpu.VMEM((_CHUNK, _D), jnp.float32),
            pltpu.VMEM((_CHUNK, _D), jnp.float32),
            pltpu.VMEM((_CHUNK, _D), jnp.float32),
            pltpu.VMEM((_CHUNK, _D), jnp.float32),
            pltpu.VMEM((_CHUNK, _D), jnp.float32),
            pltpu.VMEM((_CHUNK, _D), jnp.float32),
            pltpu.SemaphoreType.DMA,
            pltpu.SemaphoreType.DMA,
            pltpu.SemaphoreType.DMA,
            pltpu.SemaphoreType.DMA,
            pltpu.SemaphoreType.DMA,
            pltpu.SemaphoreType.DMA,
            pltpu.SemaphoreType.DMA,
            pltpu.SemaphoreType.DMA,
            pltpu.SemaphoreType.DMA,
            pltpu.SemaphoreType.DMA,
            pltpu.SemaphoreType.DMA,
            pltpu.SemaphoreType.DMA,
        ],
    )(_embed_body)
    return f(x, pe, table)


def kernel(x, table):
    return _embed(x, table).reshape(_B, _L, _D)


# speedup vs baseline: 1.9747x; 1.9747x over previous
import functools
import math

import numpy as np
import jax
import jax.numpy as jnp
from jax import lax
from jax.experimental import pallas as pl
from jax.experimental.pallas import tpu as pltpu
from jax.experimental.pallas import tpu_sc as plsc

_VOCAB = 100000
_D = 1024
_B = 4
_L = 2048
_NC, _NS = 2, 16
_NW = _NC * _NS
_PPW = _L // _NW
_ROWS = _B * _L
_CHUNK = 16
_NGROUP = _PPW // _CHUNK
_NCHUNK = _NGROUP * _B
_LANES = 16
_NBUF = 5


_NROT = _L // _CHUNK


def _pe_consts() -> np.ndarray:
    div = np.exp(np.arange(0, _D, 2).astype(np.float32)
                 * (-math.log(10000.0) / _D))
    pos = np.arange(_CHUNK, dtype=np.float32)[:, None]
    pe16 = np.zeros((_CHUNK, _D), dtype=np.float32)
    pe16[:, 0::2] = np.sin(pos * div)
    pe16[:, 1::2] = np.cos(pos * div)
    pe16s = np.zeros_like(pe16)
    pe16s[:, 0::2] = pe16[:, 1::2]
    pe16s[:, 1::2] = pe16[:, 0::2]
    k = (np.arange(_NROT, dtype=np.float32) * _CHUNK)[:, None]
    rot_a = np.zeros((_NROT, _D), dtype=np.float32)
    rot_a[:, 0::2] = np.cos(k * div)
    rot_a[:, 1::2] = rot_a[:, 0::2]
    rot_b = np.zeros((_NROT, _D), dtype=np.float32)
    rot_b[:, 0::2] = np.sin(k * div)
    rot_b[:, 1::2] = -rot_b[:, 0::2]
    return np.concatenate([pe16, pe16s, rot_a, rot_b], axis=0)


_CONSTS = _pe_consts()


def _embed_body(x_hbm, pe_hbm, table_hbm, out_hbm,
                idx_v, pv0, pv1, gb0, gb1, gb2, gb3, gb4,
                psem0, psem1, gsem0, gsem1, gsem2, gsem3, gsem4,
                ssem0, ssem1, ssem2, ssem3, ssem4):
    pe_v = (pv0, pv1)
    gbuf = (gb0, gb1, gb2, gb3, gb4)
    psem = (psem0, psem1)
    gsem = (gsem0, gsem1, gsem2, gsem3, gsem4)
    ssem = (ssem0, ssem1, ssem2, ssem3, ssem4)

    wid = lax.axis_index("s") * _NC + lax.axis_index("c")
    p0 = wid * _PPW

    for b in range(_B):
        pltpu.sync_copy(x_hbm.at[b, pl.ds(p0, _PPW)],
                        idx_v.at[pl.ds(b * _PPW, _PPW)])

    def start_gather(c):
        g, bt = divmod(c, _B)
        return pltpu.async_copy(
            table_hbm.at[idx_v.at[pl.ds(bt * _PPW + g * _CHUNK, _CHUNK)]],
            gbuf[c % _NBUF], gsem[c % _NBUF])

    def start_pe(g):
        return pltpu.async_copy(
            pe_hbm.at[pl.ds(p0 + g * _CHUNK, _CHUNK)], pe_v[g % 2],
            psem[g % 2])

    gathers = {c: start_gather(c) for c in range(_NBUF - 2)}
    pes = {0: start_pe(0), 1: start_pe(1)}
    stores = {}

    for c in range(_NCHUNK):
        g, bt = divmod(c, _B)
        if bt == 0:
            if 1 <= g <= _NGROUP - 2:
                pes[g + 1] = start_pe(g + 1)
            pes.pop(g).wait()
        gathers.pop(c).wait()

        gb, pv = gbuf[c % _NBUF], pe_v[g % 2]

        @plsc.parallel_loop(0, _CHUNK * _D // (2 * _LANES), 1, unroll=4)
        def add_body(i):
            r = lax.shift_right_logical(i, 5)
            c0 = pl.multiple_of(
                lax.bitwise_and(i, _D // (2 * _LANES) - 1) * 2 * _LANES,
                2 * _LANES)
            s1 = pl.ds(c0, _LANES)
            s2 = pl.ds(c0 + _LANES, _LANES)
            plsc.addupdate(gb.at[r, s1], pv[r, s1])
            plsc.addupdate(gb.at[r, s2], pv[r, s2])

        stores[c] = pltpu.async_copy(
            gb, out_hbm.at[pl.ds(bt * _L + p0 + g * _CHUNK, _CHUNK)],
            ssem[c % _NBUF])
        if c + _NBUF - 2 < _NCHUNK:
            if c >= 2:
                stores.pop(c - 2).wait()
            gathers[c + _NBUF - 2] = start_gather(c + _NBUF - 2)

    for c in sorted(stores):
        stores.pop(c).wait()


_KPB = 16


def _pe_tc_body(pe16_ref, pe16s_ref, ra_ref, rb_ref, o_ref):
    o_ref[...] = (pe16_ref[...][None] * ra_ref[...]
                  + pe16s_ref[...][None] * rb_ref[...])


@jax.jit
def _embed(x, table):
    consts = jnp.asarray(_CONSTS)
    pe = pl.pallas_call(
        _pe_tc_body,
        grid=(_NROT // _KPB,),
        in_specs=[
            pl.BlockSpec((_CHUNK, _D), lambda k: (0, 0)),
            pl.BlockSpec((_CHUNK, _D), lambda k: (0, 0)),
            pl.BlockSpec((_KPB, 1, _D), lambda k: (k, 0, 0)),
            pl.BlockSpec((_KPB, 1, _D), lambda k: (k, 0, 0)),
        ],
        out_specs=pl.BlockSpec((_KPB, _CHUNK, _D), lambda k: (k, 0, 0)),
        out_shape=jax.ShapeDtypeStruct((_NROT, _CHUNK, _D), jnp.float32),
    )(consts[0:_CHUNK], consts[_CHUNK:2 * _CHUNK],
      consts[2 * _CHUNK:2 * _CHUNK + _NROT].reshape(_NROT, 1, _D),
      consts[2 * _CHUNK + _NROT:].reshape(_NROT, 1, _D)).reshape(_L, _D)
    mesh = plsc.VectorSubcoreMesh(core_axis_name="c", subcore_axis_name="s")
    f = functools.partial(
        pl.kernel,
        mesh=mesh,
        out_type=jax.ShapeDtypeStruct((_ROWS, _D), jnp.float32),
        scratch_types=[
            pltpu.VMEM((_B * _PPW,), jnp.int32),
            pltpu.VMEM((_CHUNK, _D), jnp.float32),
            pltpu.VMEM((_CHUNK, _D), jnp.float32),
            pltpu.VMEM((_CHUNK, _D), jnp.float32),
            pltpu.VMEM((_CHUNK, _D), jnp.float32),
            pltpu.VMEM((_CHUNK, _D), jnp.float32),
            pltpu.VMEM((_CHUNK, _D), jnp.float32),
            pltpu.VMEM((_CHUNK, _D), jnp.float32),
            pltpu.SemaphoreType.DMA,
            pltpu.SemaphoreType.DMA,
            pltpu.SemaphoreType.DMA,
            pltpu.SemaphoreType.DMA,
            pltpu.SemaphoreType.DMA,
            pltpu.SemaphoreType.DMA,
            pltpu.SemaphoreType.DMA,
            pltpu.SemaphoreType.DMA,
            pltpu.SemaphoreType.DMA,
            pltpu.SemaphoreType.DMA,
            pltpu.SemaphoreType.DMA,
            pltpu.SemaphoreType.DMA,
        ],
    )(_embed_body)
    return f(x, pe, table)


def kernel(x, table):
    return _embed(x, table).reshape(_B, _L, _D)


# speedup vs baseline: 2.0230x; 1.0245x over previous
import functools
import math

import numpy as np
import jax
import jax.numpy as jnp
from jax import lax
from jax.experimental import pallas as pl
from jax.experimental.pallas import tpu as pltpu
from jax.experimental.pallas import tpu_sc as plsc

_VOCAB = 100000
_D = 1024
_B = 4
_L = 2048
_NC, _NS = 2, 16
_NW = _NC * _NS
_PPW = _L // _NW
_ROWS = _B * _L
_CHUNK = 16
_NGROUP = _PPW // _CHUNK
_NCHUNK = _NGROUP * _B
_LANES = 16
_NBUF = 5


_NROT = _L // _CHUNK


def _pe_consts() -> np.ndarray:
    div = np.exp(np.arange(0, _D, 2).astype(np.float32)
                 * (-math.log(10000.0) / _D))
    pos = np.arange(_CHUNK, dtype=np.float32)[:, None]
    pe16 = np.zeros((_CHUNK, _D), dtype=np.float32)
    pe16[:, 0::2] = np.sin(pos * div)
    pe16[:, 1::2] = np.cos(pos * div)
    pe16s = np.zeros_like(pe16)
    pe16s[:, 0::2] = pe16[:, 1::2]
    pe16s[:, 1::2] = pe16[:, 0::2]
    k = (np.arange(_NROT, dtype=np.float32) * _CHUNK)[:, None]
    rot_a = np.zeros((_NROT, _D), dtype=np.float32)
    rot_a[:, 0::2] = np.cos(k * div)
    rot_a[:, 1::2] = rot_a[:, 0::2]
    rot_b = np.zeros((_NROT, _D), dtype=np.float32)
    rot_b[:, 0::2] = np.sin(k * div)
    rot_b[:, 1::2] = -rot_b[:, 0::2]
    return np.concatenate([pe16, pe16s, rot_a, rot_b], axis=0)


_CONSTS = _pe_consts()


def _embed_body(x_hbm, pe_hbm, table_hbm, out_hbm,
                idx_v, pv0, pv1, gb0, gb1, gb2, gb3, gb4,
                psem0, psem1, gsem0, gsem1, gsem2, gsem3, gsem4,
                ssem0, ssem1, ssem2, ssem3, ssem4):
    pe_v = (pv0, pv1)
    gbuf = (gb0, gb1, gb2, gb3, gb4)
    psem = (psem0, psem1)
    gsem = (gsem0, gsem1, gsem2, gsem3, gsem4)
    ssem = (ssem0, ssem1, ssem2, ssem3, ssem4)

    wid = lax.axis_index("s") * _NC + lax.axis_index("c")
    p0 = wid * _PPW

    for b in range(_B):
        pltpu.sync_copy(x_hbm.at[b, pl.ds(p0, _PPW)],
                        idx_v.at[pl.ds(b * _PPW, _PPW)])

    def start_gather(c):
        g, bt = divmod(c, _B)
        return pltpu.async_copy(
            table_hbm.at[idx_v.at[pl.ds(bt * _PPW + g * _CHUNK, _CHUNK)]],
            gbuf[c % _NBUF], gsem[c % _NBUF])

    def start_pe(g):
        return pltpu.async_copy(
            pe_hbm.at[pl.ds(p0 + g * _CHUNK, _CHUNK)], pe_v[g % 2],
            psem[g % 2])

    gathers = {c: start_gather(c) for c in range(_NBUF - 2)}
    pes = {0: start_pe(0), 1: start_pe(1)}
    stores = {}

    for c in range(_NCHUNK):
        g, bt = divmod(c, _B)
        if bt == 0:
            if 1 <= g <= _NGROUP - 2:
                pes[g + 1] = start_pe(g + 1)
            pes.pop(g).wait()
        gathers.pop(c).wait()

        gb, pv = gbuf[c % _NBUF], pe_v[g % 2]

        @plsc.parallel_loop(0, _CHUNK * _D // (2 * _LANES), 1, unroll=4)
        def add_body(i):
            r = lax.shift_right_logical(i, 5)
            c0 = pl.multiple_of(
                lax.bitwise_and(i, _D // (2 * _LANES) - 1) * 2 * _LANES,
                2 * _LANES)
            s1 = pl.ds(c0, _LANES)
            s2 = pl.ds(c0 + _LANES, _LANES)
            plsc.addupdate(gb.at[r, s1], pv[r, s1])
            plsc.addupdate(gb.at[r, s2], pv[r, s2])

        stores[c] = pltpu.async_copy(
            gb, out_hbm.at[pl.ds(bt * _L + p0 + g * _CHUNK, _CHUNK)],
            ssem[c % _NBUF])
        if c + _NBUF - 2 < _NCHUNK:
            if c >= 2:
                stores.pop(c - 2).wait()
            gathers[c + _NBUF - 2] = start_gather(c + _NBUF - 2)

    for c in sorted(stores):
        stores.pop(c).wait()


_KPB = 32


def _pe_tc_body(pe16_ref, pe16s_ref, ra_ref, rb_ref, o_ref):
    o_ref[...] = (pe16_ref[...][None] * ra_ref[...]
                  + pe16s_ref[...][None] * rb_ref[...])


@jax.jit
def _embed(x, table):
    consts = jnp.asarray(_CONSTS)
    pe = pl.pallas_call(
        _pe_tc_body,
        grid=(_NROT // _KPB,),
        in_specs=[
            pl.BlockSpec((_CHUNK, _D), lambda k: (0, 0)),
            pl.BlockSpec((_CHUNK, _D), lambda k: (0, 0)),
            pl.BlockSpec((_KPB, 1, _D), lambda k: (k, 0, 0)),
            pl.BlockSpec((_KPB, 1, _D), lambda k: (k, 0, 0)),
        ],
        out_specs=pl.BlockSpec((_KPB, _CHUNK, _D), lambda k: (k, 0, 0)),
        out_shape=jax.ShapeDtypeStruct((_NROT, _CHUNK, _D), jnp.float32),
    )(consts[0:_CHUNK], consts[_CHUNK:2 * _CHUNK],
      consts[2 * _CHUNK:2 * _CHUNK + _NROT].reshape(_NROT, 1, _D),
      consts[2 * _CHUNK + _NROT:].reshape(_NROT, 1, _D)).reshape(_L, _D)
    mesh = plsc.VectorSubcoreMesh(core_axis_name="c", subcore_axis_name="s")
    f = functools.partial(
        pl.kernel,
        mesh=mesh,
        out_type=jax.ShapeDtypeStruct((_ROWS, _D), jnp.float32),
        scratch_types=[
            pltpu.VMEM((_B * _PPW,), jnp.int32),
            pltpu.VMEM((_CHUNK, _D), jnp.float32),
            pltpu.VMEM((_CHUNK, _D), jnp.float32),
            pltpu.VMEM((_CHUNK, _D), jnp.float32),
            pltpu.VMEM((_CHUNK, _D), jnp.float32),
            pltpu.VMEM((_CHUNK, _D), jnp.float32),
            pltpu.VMEM((_CHUNK, _D), jnp.float32),
            pltpu.VMEM((_CHUNK, _D), jnp.float32),
            pltpu.SemaphoreType.DMA,
            pltpu.SemaphoreType.DMA,
            pltpu.SemaphoreType.DMA,
            pltpu.SemaphoreType.DMA,
            pltpu.SemaphoreType.DMA,
            pltpu.SemaphoreType.DMA,
            pltpu.SemaphoreType.DMA,
            pltpu.SemaphoreType.DMA,
            pltpu.SemaphoreType.DMA,
            pltpu.SemaphoreType.DMA,
            pltpu.SemaphoreType.DMA,
            pltpu.SemaphoreType.DMA,
        ],
    )(_embed_body)
    return f(x, pe, table)


def kernel(x, table):
    return _embed(x, table).reshape(_B, _L, _D)


# speedup vs baseline: 2.0540x; 1.0153x over previous
import functools
import math

import numpy as np
import jax
import jax.numpy as jnp
from jax import lax
from jax.experimental import pallas as pl
from jax.experimental.pallas import tpu as pltpu
from jax.experimental.pallas import tpu_sc as plsc

_VOCAB = 100000
_D = 1024
_B = 4
_L = 2048
_NC, _NS = 2, 16
_NW = _NC * _NS
_PPW = _L // _NW
_ROWS = _B * _L
_CHUNK = 16
_NGROUP = _PPW // _CHUNK
_NCHUNK = _NGROUP * _B
_LANES = 16
_NBUF = 5


_NROT = _L // _CHUNK


def _pe_consts() -> np.ndarray:
    div = np.exp(np.arange(0, _D, 2).astype(np.float32)
                 * (-math.log(10000.0) / _D))
    pos = np.arange(_CHUNK, dtype=np.float32)[:, None]
    pe16 = np.zeros((_CHUNK, _D), dtype=np.float32)
    pe16[:, 0::2] = np.sin(pos * div)
    pe16[:, 1::2] = np.cos(pos * div)
    pe16s = np.zeros_like(pe16)
    pe16s[:, 0::2] = pe16[:, 1::2]
    pe16s[:, 1::2] = pe16[:, 0::2]
    k = (np.arange(_NROT, dtype=np.float32) * _CHUNK)[:, None]
    rot_a = np.zeros((_NROT, _D), dtype=np.float32)
    rot_a[:, 0::2] = np.cos(k * div)
    rot_a[:, 1::2] = rot_a[:, 0::2]
    rot_b = np.zeros((_NROT, _D), dtype=np.float32)
    rot_b[:, 0::2] = np.sin(k * div)
    rot_b[:, 1::2] = -rot_b[:, 0::2]
    return np.concatenate([pe16, pe16s, rot_a, rot_b], axis=0)


_CONSTS = _pe_consts()


def _embed_body(x_hbm, pe_hbm, table_hbm, out_hbm,
                idx_v, pv0, pv1, gb0, gb1, gb2, gb3, gb4,
                psem0, psem1, gsem0, gsem1, gsem2, gsem3, gsem4,
                ssem0, ssem1, ssem2, ssem3, ssem4):
    pe_v = (pv0, pv1)
    gbuf = (gb0, gb1, gb2, gb3, gb4)
    psem = (psem0, psem1)
    gsem = (gsem0, gsem1, gsem2, gsem3, gsem4)
    ssem = (ssem0, ssem1, ssem2, ssem3, ssem4)

    wid = lax.axis_index("s") * _NC + lax.axis_index("c")
    p0 = wid * _PPW

    for b in range(_B):
        pltpu.sync_copy(x_hbm.at[b, pl.ds(p0, _PPW)],
                        idx_v.at[pl.ds(b * _PPW, _PPW)])

    def start_gather(c):
        g, bt = divmod(c, _B)
        return pltpu.async_copy(
            table_hbm.at[idx_v.at[pl.ds(bt * _PPW + g * _CHUNK, _CHUNK)]],
            gbuf[c % _NBUF], gsem[c % _NBUF])

    def start_pe(g):
        return pltpu.async_copy(
            pe_hbm.at[pl.ds(p0 + g * _CHUNK, _CHUNK)], pe_v[g % 2],
            psem[g % 2])

    gathers = {c: start_gather(c) for c in range(_NBUF - 2)}
    pes = {0: start_pe(0), 1: start_pe(1)}
    stores = {}

    for c in range(_NCHUNK):
        g, bt = divmod(c, _B)
        if bt == 0:
            if 1 <= g <= _NGROUP - 2:
                pes[g + 1] = start_pe(g + 1)
            pes.pop(g).wait()
        gathers.pop(c).wait()

        gb, pv = gbuf[c % _NBUF], pe_v[g % 2]

        @plsc.parallel_loop(0, _CHUNK * _D // (2 * _LANES), 1, unroll=4)
        def add_body(i):
            r = lax.shift_right_logical(i, 5)
            c0 = pl.multiple_of(
                lax.bitwise_and(i, _D // (2 * _LANES) - 1) * 2 * _LANES,
                2 * _LANES)
            s1 = pl.ds(c0, _LANES)
            s2 = pl.ds(c0 + _LANES, _LANES)
            plsc.addupdate(gb.at[r, s1], pv[r, s1])
            plsc.addupdate(gb.at[r, s2], pv[r, s2])

        stores[c] = pltpu.async_copy(
            gb, out_hbm.at[pl.ds(bt * _L + p0 + g * _CHUNK, _CHUNK)],
            ssem[c % _NBUF])
        if c + _NBUF - 2 < _NCHUNK:
            if c >= 2:
                stores.pop(c - 2).wait()
            gathers[c + _NBUF - 2] = start_gather(c + _NBUF - 2)

    for c in sorted(stores):
        stores.pop(c).wait()


_KPB = 64


def _pe_tc_body(pe16_ref, pe16s_ref, ra_ref, rb_ref, o_ref):
    o_ref[...] = (pe16_ref[...][None] * ra_ref[...]
                  + pe16s_ref[...][None] * rb_ref[...])


@jax.jit
def _embed(x, table):
    consts = jnp.asarray(_CONSTS)
    pe = pl.pallas_call(
        _pe_tc_body,
        grid=(_NROT // _KPB,),
        in_specs=[
            pl.BlockSpec((_CHUNK, _D), lambda k: (0, 0)),
            pl.BlockSpec((_CHUNK, _D), lambda k: (0, 0)),
            pl.BlockSpec((_KPB, 1, _D), lambda k: (k, 0, 0)),
            pl.BlockSpec((_KPB, 1, _D), lambda k: (k, 0, 0)),
        ],
        out_specs=pl.BlockSpec((_KPB, _CHUNK, _D), lambda k: (k, 0, 0)),
        out_shape=jax.ShapeDtypeStruct((_NROT, _CHUNK, _D), jnp.float32),
    )(consts[0:_CHUNK], consts[_CHUNK:2 * _CHUNK],
      consts[2 * _CHUNK:2 * _CHUNK + _NROT].reshape(_NROT, 1, _D),
      consts[2 * _CHUNK + _NROT:].reshape(_NROT, 1, _D)).reshape(_L, _D)
    mesh = plsc.VectorSubcoreMesh(core_axis_name="c", subcore_axis_name="s")
    f = functools.partial(
        pl.kernel,
        mesh=mesh,
        out_type=jax.ShapeDtypeStruct((_ROWS, _D), jnp.float32),
        scratch_types=[
            pltpu.VMEM((_B * _PPW,), jnp.int32),
            pltpu.VMEM((_CHUNK, _D), jnp.float32),
            pltpu.VMEM((_CHUNK, _D), jnp.float32),
            pltpu.VMEM((_CHUNK, _D), jnp.float32),
            pltpu.VMEM((_CHUNK, _D), jnp.float32),
            pltpu.VMEM((_CHUNK, _D), jnp.float32),
            pltpu.VMEM((_CHUNK, _D), jnp.float32),
            pltpu.VMEM((_CHUNK, _D), jnp.float32),
            pltpu.SemaphoreType.DMA,
            pltpu.SemaphoreType.DMA,
            pltpu.SemaphoreType.DMA,
            pltpu.SemaphoreType.DMA,
            pltpu.SemaphoreType.DMA,
            pltpu.SemaphoreType.DMA,
            pltpu.SemaphoreType.DMA,
            pltpu.SemaphoreType.DMA,
            pltpu.SemaphoreType.DMA,
            pltpu.SemaphoreType.DMA,
            pltpu.SemaphoreType.DMA,
            pltpu.SemaphoreType.DMA,
        ],
    )(_embed_body)
    return f(x, pe, table)


def kernel(x, table):
    return _embed(x, table).reshape(_B, _L, _D)
